# baseline (speedup 1.0000x reference)
import functools

import jax
import jax.numpy as jnp
from jax import lax
from jax.experimental import pallas as pl
from jax.experimental.pallas import tpu as pltpu
from jax.experimental.pallas import tpu_sc as plsc

N_NODES = 10000
N_EDGES = 320000
RBF = 128
EMB = 32
PROJ = 64
OUT = 64

NODE_BLK = 1000
EDGE_BLK = 8000

F32 = jnp.float32


def _node_emb_body(el_ref, ch_ref, nh_ref, hy_ref,
                   ael_ref, ach_ref, anh_ref, ahy_ref, bn_ref, out_ref):
    bn = el_ref.shape[0]
    iota = lax.broadcasted_iota(jnp.int32, (bn, 128), 1)

    def oh_dot(idx_ref, tab_ref):
        oh = (idx_ref[...] == iota).astype(F32)
        return jnp.dot(oh, tab_ref[...], preferred_element_type=F32)

    out_ref[...] = (oh_dot(el_ref, ael_ref) + oh_dot(ch_ref, ach_ref)
                    + oh_dot(nh_ref, anh_ref) + oh_dot(hy_ref, ahy_ref)
                    + bn_ref[...])


def _node_emb_call(el, ch, nh, hy, a_el, a_ch, a_nh, a_hy, bn_row,
                   interpret=False):
    grid = (N_NODES // NODE_BLK,)
    idx_spec = pl.BlockSpec((NODE_BLK, 1), lambda i: (i, 0))
    full = lambda shape: pl.BlockSpec(shape, lambda i: (0, 0))
    return pl.pallas_call(
        _node_emb_body,
        grid=grid,
        in_specs=[idx_spec, idx_spec, idx_spec, idx_spec,
                  full((128, EMB)), full((128, EMB)), full((128, EMB)),
                  full((128, EMB)), full((1, EMB))],
        out_specs=pl.BlockSpec((NODE_BLK, EMB), lambda i: (i, 0)),
        out_shape=jax.ShapeDtypeStruct((N_NODES, EMB), F32),
        interpret=interpret,
    )(el, ch, nh, hy, a_el, a_ch, a_nh, a_hy, bn_row)


_NW = 32
_PER_W = N_EDGES // _NW
_CH = 80
_NCH = _PER_W // _CH
_GW = 128

@functools.cache
def _make_sc_gather():
    mesh = plsc.VectorSubcoreMesh(core_axis_name="c", subcore_axis_name="s",
                                  num_cores=2, num_subcores=16)

    @functools.partial(
        pl.kernel,
        out_type=(jax.ShapeDtypeStruct((N_EDGES, EMB), F32),
                  jax.ShapeDtypeStruct((N_EDGES, EMB), F32),
                  jax.ShapeDtypeStruct((N_EDGES, PROJ), F32)),
        mesh=mesh,
        scratch_types=[
            pltpu.VMEM((_PER_W,), jnp.int32),
            pltpu.VMEM((_PER_W,), jnp.int32),
            pltpu.VMEM((_PER_W,), jnp.int32),
            pltpu.VMEM((_CH, EMB), F32),
            pltpu.VMEM((_CH, EMB), F32),
            pltpu.VMEM((_CH, PROJ), F32),
            pltpu.VMEM((_CH, EMB), F32),
            pltpu.VMEM((_CH, EMB), F32),
            pltpu.VMEM((_CH, PROJ), F32),
            pltpu.SemaphoreType.DMA,
            pltpu.SemaphoreType.DMA,
        ],
        compiler_params=pltpu.CompilerParams(use_tc_tiling_on_sc=False),
    )
    def _sc_gather(ne_hbm, src_hbm, dst_hbm, s_hbm, tsm_hbm,
                   es_hbm, ed_hbm, ts_hbm,
                   idx_s, idx_d, idx_t,
                   rs0, rd0, rt0, rs1, rd1, rt1, sem0, sem1):
        wid = lax.axis_index("s") * 2 + lax.axis_index("c")
        base = pl.multiple_of(wid * _PER_W, 8)
        pltpu.sync_copy(src_hbm.at[pl.ds(base, _PER_W)], idx_s)
        pltpu.sync_copy(dst_hbm.at[pl.ds(base, _PER_W)], idx_d)
        pltpu.sync_copy(s_hbm.at[pl.ds(base, _PER_W)], idx_t)

        def issue(i, rs, rd, rt, sem):
            off = pl.multiple_of(i * _CH, 8)
            pltpu.async_copy(ne_hbm.at[idx_s.at[pl.ds(off, _CH)]], rs, sem)
            pltpu.async_copy(ne_hbm.at[idx_d.at[pl.ds(off, _CH)]], rd, sem)
            pltpu.async_copy(tsm_hbm.at[idx_t.at[pl.ds(off, _CH)]], rt, sem)

        def drain_wb(i, rs, rd, rt, sem):
            pltpu.make_async_copy(ne_hbm.at[pl.ds(0, _CH)], rs, sem).wait()
            pltpu.make_async_copy(ne_hbm.at[pl.ds(0, _CH)], rd, sem).wait()
            pltpu.make_async_copy(ts_hbm.at[pl.ds(0, _CH)], rt, sem).wait()
            gbase = pl.multiple_of(base + i * _CH, 8)
            pltpu.sync_copy(rs, es_hbm.at[pl.ds(gbase, _CH)])
            pltpu.sync_copy(rd, ed_hbm.at[pl.ds(gbase, _CH)])
            pltpu.sync_copy(rt, ts_hbm.at[pl.ds(gbase, _CH)])

        issue(0, rs0, rd0, rt0, sem0)

        def body(k, carry):
            i0 = 2 * k
            issue(i0 + 1, rs1, rd1, rt1, sem1)
            drain_wb(i0, rs0, rd0, rt0, sem0)
            issue(i0 + 2, rs0, rd0, rt0, sem0)
            drain_wb(i0 + 1, rs1, rd1, rt1, sem1)
            return carry

        lax.fori_loop(0, (_NCH - 1) // 2, body, 0)
        drain_wb(_NCH - 1, rs0, rd0, rt0, sem0)

    return _sc_gather


def _unpack(packed, k, width, bsz):
    parts = [packed[:, None, j * width:(j + 1) * width] for j in range(k)]
    return jnp.concatenate(parts, axis=1).reshape(bsz, width)


def _edge_body(r_ref, es4_ref, ed4_ref, ts2_ref, w1r_ref, w1s_ref, w1d_ref,
               gam_ref, bet_ref, w2_ref, b2_ref, out_ref):
    bsz = r_ref.shape[0]
    es = _unpack(es4_ref[...], 4, EMB, bsz)
    ed = _unpack(ed4_ref[...], 4, EMB, bsz)
    ts = _unpack(ts2_ref[...], 2, PROJ, bsz)
    h = (jnp.dot(r_ref[...], w1r_ref[...], preferred_element_type=F32)
         + jnp.dot(es, w1s_ref[...], preferred_element_type=F32)
         + jnp.dot(ed, w1d_ref[...], preferred_element_type=F32)
         + ts)
    m = jnp.mean(h, axis=-1, keepdims=True)
    v = jnp.mean((h - m) * (h - m), axis=-1, keepdims=True)
    hn = (h - m) * lax.rsqrt(v + 1e-5) * gam_ref[...] + bet_ref[...]
    hs = hn * lax.logistic(hn)
    res = (jnp.dot(hs, w2_ref[...], preferred_element_type=F32)
           + b2_ref[...])
    r3 = res.reshape(bsz // 2, 2, OUT)
    out_ref[...] = jnp.concatenate([r3[:, 0, :], r3[:, 1, :]], axis=1)


def _edge_call(R, es4, ed4, ts2, w1r, w1s, w1d, gam, bet, w2, b2,
               interpret=False):
    grid = (N_EDGES // EDGE_BLK,)
    full = lambda shape: pl.BlockSpec(shape, lambda i: (0, 0))
    return pl.pallas_call(
        _edge_body,
        grid=grid,
        in_specs=[pl.BlockSpec((EDGE_BLK, RBF), lambda i: (i, 0)),
                  pl.BlockSpec((EDGE_BLK // 4, 128), lambda i: (i, 0)),
                  pl.BlockSpec((EDGE_BLK // 4, 128), lambda i: (i, 0)),
                  pl.BlockSpec((EDGE_BLK // 2, 128), lambda i: (i, 0)),
                  full((RBF, PROJ)), full((EMB, PROJ)), full((EMB, PROJ)),
                  full((1, PROJ)), full((1, PROJ)),
                  full((PROJ, OUT)), full((1, OUT))],
        out_specs=pl.BlockSpec((EDGE_BLK // 2, 128), lambda i: (i, 0)),
        out_shape=jax.ShapeDtypeStruct((N_EDGES // 2, 128), F32),
        interpret=interpret,
    )(R, es4, ed4, ts2, w1r, w1s, w1d, gam, bet, w2, b2)


def _fold_weights(tb_bond, tb_ring, tb_arom, We, be,
                  t_el, t_ch, t_nh, t_hy, Wn, bn, W1, b1):
    def pad128(a):
        return jnp.zeros((128, EMB), F32).at[:a.shape[0]].set(a)

    a_el = pad128(t_el @ Wn[0:32])
    a_ch = pad128(t_ch @ Wn[32:64])
    a_nh = pad128(t_nh @ Wn[64:96])
    a_hy = pad128(t_hy @ Wn[96:128])

    cat = ((tb_bond @ We[0:32])[:, None, None, :]
           + (tb_ring @ We[32:64])[None, :, None, :]
           + (tb_arom @ We[64:96])[None, None, :, :]
           + be)
    t_small = (cat.reshape(32, EMB) @ W1[RBF:RBF + EMB]) + b1

    w1r = W1[:RBF]
    w1s = W1[RBF + EMB:RBF + 2 * EMB]
    w1d = W1[RBF + 2 * EMB:RBF + 3 * EMB]
    return a_el, a_ch, a_nh, a_hy, t_small, w1r, w1s, w1d


def kernel(R, edge_index, bond_order, is_in_ring, is_aromatic, element,
           charge, nhyd, hyb, tb_bond, tb_ring, tb_arom, We, be, t_el, t_ch,
           t_nh, t_hy, Wn, bn, W1, b1, ln_g, ln_b, W2, b2):
    a_el, a_ch, a_nh, a_hy, t_small, w1r, w1s, w1d = _fold_weights(
        tb_bond, tb_ring, tb_arom, We, be, t_el, t_ch, t_nh, t_hy, Wn, bn,
        W1, b1)

    col = lambda a: a.reshape(-1, 1).astype(jnp.int32)
    ne = _node_emb_call(col(element), col(charge), col(nhyd), col(hyb),
                        a_el, a_ch, a_nh, a_hy, bn.reshape(1, EMB))

    src = edge_index[:, 0].astype(jnp.int32)
    dst = edge_index[:, 1].astype(jnp.int32)
    s = (bond_order * 4 + is_in_ring * 2 + is_aromatic).astype(jnp.int32)
    es, ed, ts = _make_sc_gather()(ne, src, dst, s, t_small)

    out = _edge_call(R, es.reshape(N_EDGES // 4, 128),
                     ed.reshape(N_EDGES // 4, 128),
                     ts.reshape(N_EDGES // 2, 128),
                     w1r, w1s, w1d, ln_g.reshape(1, PROJ),
                     ln_b.reshape(1, PROJ), W2, b2.reshape(1, OUT))
    return out.reshape(N_EDGES, OUT)


# speedup vs baseline: 3.5051x; 3.5051x over previous
import functools

import jax
import jax.numpy as jnp
from jax import lax
from jax.experimental import pallas as pl
from jax.experimental.pallas import tpu as pltpu
from jax.experimental.pallas import tpu_sc as plsc

N_NODES = 10000
N_EDGES = 320000
RBF = 128
EMB = 32
PROJ = 64
OUT = 64

NODE_BLK = 1000
EDGE_BLK = 8000

F32 = jnp.float32


def _node_emb_body(el_ref, ch_ref, nh_ref, hy_ref,
                   ael_ref, ach_ref, anh_ref, ahy_ref, bn_ref, out_ref):
    bn = el_ref.shape[0]
    iota = lax.broadcasted_iota(jnp.int32, (bn, 128), 1)

    def oh_dot(idx_ref, tab_ref):
        oh = (idx_ref[...] == iota).astype(F32)
        return jnp.dot(oh, tab_ref[...], preferred_element_type=F32)

    out_ref[...] = (oh_dot(el_ref, ael_ref) + oh_dot(ch_ref, ach_ref)
                    + oh_dot(nh_ref, anh_ref) + oh_dot(hy_ref, ahy_ref)
                    + bn_ref[...])


def _node_emb_call(el, ch, nh, hy, a_el, a_ch, a_nh, a_hy, bn_row,
                   interpret=False):
    grid = (N_NODES // NODE_BLK,)
    idx_spec = pl.BlockSpec((NODE_BLK, 1), lambda i: (i, 0))
    full = lambda shape: pl.BlockSpec(shape, lambda i: (0, 0))
    return pl.pallas_call(
        _node_emb_body,
        grid=grid,
        in_specs=[idx_spec, idx_spec, idx_spec, idx_spec,
                  full((128, EMB)), full((128, EMB)), full((128, EMB)),
                  full((128, EMB)), full((1, EMB))],
        out_specs=pl.BlockSpec((NODE_BLK, EMB), lambda i: (i, 0)),
        out_shape=jax.ShapeDtypeStruct((N_NODES, EMB), F32),
        interpret=interpret,
    )(el, ch, nh, hy, a_el, a_ch, a_nh, a_hy, bn_row)


_NW = 32
_PER_W = N_EDGES // _NW
_CH = 80
_NCH = _PER_W // _CH
_GW = 128

@functools.cache
def _make_sc_gather():
    mesh = plsc.VectorSubcoreMesh(core_axis_name="c", subcore_axis_name="s",
                                  num_cores=2, num_subcores=16)

    @functools.partial(
        pl.kernel,
        out_type=jax.ShapeDtypeStruct((N_EDGES, _GW), F32),
        mesh=mesh,
        scratch_types=[
            pltpu.VMEM((_PER_W,), jnp.int32),
            pltpu.VMEM((_PER_W,), jnp.int32),
            pltpu.VMEM((_PER_W,), jnp.int32),
            pltpu.VMEM((_CH, EMB), F32),
            pltpu.VMEM((_CH, EMB), F32),
            pltpu.VMEM((_CH, PROJ), F32),
            pltpu.VMEM((_CH, EMB), F32),
            pltpu.VMEM((_CH, EMB), F32),
            pltpu.VMEM((_CH, PROJ), F32),
            pltpu.SemaphoreType.DMA,
            pltpu.SemaphoreType.DMA,
            pltpu.SemaphoreType.DMA,
            pltpu.SemaphoreType.DMA,
            pltpu.SemaphoreType.DMA,
            pltpu.SemaphoreType.DMA,
        ],
        compiler_params=pltpu.CompilerParams(use_tc_tiling_on_sc=False),
    )
    def _sc_gather(ne_hbm, src_hbm, dst_hbm, s_hbm, tsm_hbm, g_hbm,
                   idx_s, idx_d, idx_t,
                   rs0, rd0, rt0, rs1, rd1, rt1,
                   ss0, sd0, st0, ss1, sd1, st1):
        wid = lax.axis_index("s") * 2 + lax.axis_index("c")
        base = pl.multiple_of(wid * _PER_W, 8)
        pltpu.sync_copy(src_hbm.at[pl.ds(base, _PER_W)], idx_s)
        pltpu.sync_copy(dst_hbm.at[pl.ds(base, _PER_W)], idx_d)
        pltpu.sync_copy(s_hbm.at[pl.ds(base, _PER_W)], idx_t)

        def issue(i, rs, rd, rt, ss, sd, st):
            off = pl.multiple_of(i * _CH, 8)
            pltpu.async_copy(ne_hbm.at[idx_s.at[pl.ds(off, _CH)]], rs, ss)
            pltpu.async_copy(ne_hbm.at[idx_d.at[pl.ds(off, _CH)]], rd, sd)
            pltpu.async_copy(tsm_hbm.at[idx_t.at[pl.ds(off, _CH)]], rt, st)

        def drain_wb(i, rs, rd, rt, ss, sd, st):
            pltpu.make_async_copy(ne_hbm.at[pl.ds(0, _CH)], rs, ss).wait()
            pltpu.make_async_copy(ne_hbm.at[pl.ds(0, _CH)], rd, sd).wait()
            pltpu.make_async_copy(
                g_hbm.at[pl.ds(0, _CH), pl.ds(0, PROJ)], rt, st).wait()
            gbase = pl.multiple_of(base + i * _CH, 8)
            rows = g_hbm.at[pl.ds(gbase, _CH)]
            pltpu.sync_copy(rs, rows.at[:, pl.ds(0, EMB)])
            pltpu.sync_copy(rd, rows.at[:, pl.ds(EMB, EMB)])
            pltpu.sync_copy(rt, rows.at[:, pl.ds(2 * EMB, PROJ)])

        issue(0, rs0, rd0, rt0, ss0, sd0, st0)

        def body(k, carry):
            i0 = 2 * k
            issue(i0 + 1, rs1, rd1, rt1, ss1, sd1, st1)
            drain_wb(i0, rs0, rd0, rt0, ss0, sd0, st0)
            issue(i0 + 2, rs0, rd0, rt0, ss0, sd0, st0)
            drain_wb(i0 + 1, rs1, rd1, rt1, ss1, sd1, st1)
            return carry

        lax.fori_loop(0, (_NCH - 1) // 2, body, 0)
        drain_wb(_NCH - 1, rs0, rd0, rt0, ss0, sd0, st0)

    return _sc_gather


def _edge_body(r_ref, g_ref, w1r_ref, wg_ref, gam_ref, bet_ref,
               w2_ref, b2_ref, out_ref):
    h = (jnp.dot(r_ref[...], w1r_ref[...], preferred_element_type=F32)
         + jnp.dot(g_ref[...], wg_ref[...], preferred_element_type=F32))
    m = jnp.mean(h, axis=-1, keepdims=True)
    v = jnp.mean((h - m) * (h - m), axis=-1, keepdims=True)
    hn = (h - m) * lax.rsqrt(v + 1e-5) * gam_ref[...] + bet_ref[...]
    hs = hn * lax.logistic(hn)
    out_ref[...] = (jnp.dot(hs, w2_ref[...], preferred_element_type=F32)
                    + b2_ref[...])


def _edge_call(R, G, w1r, wg, gam, bet, w2, b2, interpret=False):
    grid = (N_EDGES // EDGE_BLK,)
    row = lambda w: pl.BlockSpec((EDGE_BLK, w), lambda i: (i, 0))
    full = lambda shape: pl.BlockSpec(shape, lambda i: (0, 0))
    return pl.pallas_call(
        _edge_body,
        grid=grid,
        in_specs=[row(RBF), row(_GW),
                  full((RBF, PROJ)), full((_GW, PROJ)),
                  full((1, PROJ)), full((1, PROJ)),
                  full((PROJ, OUT)), full((1, OUT))],
        out_specs=pl.BlockSpec((EDGE_BLK, OUT), lambda i: (i, 0)),
        out_shape=jax.ShapeDtypeStruct((N_EDGES, OUT), F32),
        interpret=interpret,
    )(R, G, w1r, wg, gam, bet, w2, b2)


def _fold_weights(tb_bond, tb_ring, tb_arom, We, be,
                  t_el, t_ch, t_nh, t_hy, Wn, bn, W1, b1):
    def pad128(a):
        return jnp.zeros((128, EMB), F32).at[:a.shape[0]].set(a)

    a_el = pad128(t_el @ Wn[0:32])
    a_ch = pad128(t_ch @ Wn[32:64])
    a_nh = pad128(t_nh @ Wn[64:96])
    a_hy = pad128(t_hy @ Wn[96:128])

    cat = ((tb_bond @ We[0:32])[:, None, None, :]
           + (tb_ring @ We[32:64])[None, :, None, :]
           + (tb_arom @ We[64:96])[None, None, :, :]
           + be)
    t_small = (cat.reshape(32, EMB) @ W1[RBF:RBF + EMB]) + b1

    w1r = W1[:RBF]
    w1s = W1[RBF + EMB:RBF + 2 * EMB]
    w1d = W1[RBF + 2 * EMB:RBF + 3 * EMB]
    wg = jnp.concatenate([w1s, w1d, jnp.eye(PROJ, dtype=F32)], axis=0)
    return a_el, a_ch, a_nh, a_hy, t_small, w1r, wg


def kernel(R, edge_index, bond_order, is_in_ring, is_aromatic, element,
           charge, nhyd, hyb, tb_bond, tb_ring, tb_arom, We, be, t_el, t_ch,
           t_nh, t_hy, Wn, bn, W1, b1, ln_g, ln_b, W2, b2):
    a_el, a_ch, a_nh, a_hy, t_small, w1r, wg = _fold_weights(
        tb_bond, tb_ring, tb_arom, We, be, t_el, t_ch, t_nh, t_hy, Wn, bn,
        W1, b1)

    col = lambda a: a.reshape(-1, 1).astype(jnp.int32)
    ne = _node_emb_call(col(element), col(charge), col(nhyd), col(hyb),
                        a_el, a_ch, a_nh, a_hy, bn.reshape(1, EMB))

    src = edge_index[:, 0].astype(jnp.int32)
    dst = edge_index[:, 1].astype(jnp.int32)
    s = (bond_order * 4 + is_in_ring * 2 + is_aromatic
         + 32 * (jnp.arange(N_EDGES) // _PER_W)).astype(jnp.int32)
    tsm_rep = jnp.tile(t_small, (_NW, 1))
    G = _make_sc_gather()(ne, src, dst, s, tsm_rep)

    return _edge_call(R, G, w1r, wg, ln_g.reshape(1, PROJ),
                      ln_b.reshape(1, PROJ), W2, b2.reshape(1, OUT))
